# Initial kernel scaffold; baseline (speedup 1.0000x reference)
import jax
import jax.numpy as jnp
from jax.experimental import pallas as pl


def kernel(pred, target, weight):
    raise NotImplementedError("write your pallas kernel here")


# speedup vs baseline: 5.5627x; 5.5627x over previous
import functools

import jax
import jax.numpy as jnp
from jax import lax
from jax.experimental import pallas as pl
from jax.experimental.pallas import tpu as pltpu
from jax.experimental.pallas import tpu_sc as plsc

B = 4
N = 65536
M = N * 3
GRP = 8
CH = M // GRP
PCH = N // GRP
L = 16
NV = CH // L
UN = 8
EPS = 1e-07
_MASK31 = 0x7FFFFFFF


def _sc_point_loss(pred_f, target_f, weight_f):
    mesh = plsc.VectorSubcoreMesh(core_axis_name="c", subcore_axis_name="s")

    @functools.partial(
        pl.kernel,
        mesh=mesh,
        out_type=jax.ShapeDtypeStruct((B * L,), jnp.float32),
        compiler_params=pltpu.CompilerParams(needs_layout_passes=False),
        scratch_types=[
            pltpu.VMEM((CH,), jnp.float32),
            pltpu.VMEM((CH,), jnp.float32),
            pltpu.VMEM((PCH,), jnp.float32),
            pltpu.VMEM((CH,), jnp.int32),
            pltpu.VMEM((CH,), jnp.float32),
            pltpu.VMEM((L,), jnp.float32),
            pltpu.VMEM((GRP * L,), jnp.float32),
            pltpu.VMEM((L,), jnp.float32),
            pltpu.VMEM_SHARED((2, GRP * L), jnp.float32),
        ],
    )
    def k(pred_hbm, target_hbm, weight_hbm, out_hbm,
          p_v, t_v, w_v, key_v, wx_v, stage_v, grp_v, out_v, shared):
        cid = lax.axis_index("c")
        sid = lax.axis_index("s")
        g = sid // GRP
        lid = sid % GRP
        b = cid * 2 + g
        lane = lax.iota(jnp.int32, L)

        pltpu.sync_copy(pred_hbm.at[pl.ds(b * M + lid * CH, CH)], p_v)
        pltpu.sync_copy(target_hbm.at[pl.ds(b * M + lid * CH, CH)], t_v)
        pltpu.sync_copy(weight_hbm.at[pl.ds(b * N + lid * PCH, PCH)], w_v)

        zero = jnp.zeros((L,), jnp.float32)
        eps = jnp.float32(EPS)

        def global_sum(vec):
            stage_v[...] = vec
            plsc.subcore_barrier()
            pltpu.sync_copy(stage_v, shared.at[g, pl.ds(lid * L, L)])
            plsc.subcore_barrier()
            pltpu.sync_copy(shared.at[g], grp_v)

            def rd(j, acc):
                return acc + grp_v[pl.ds(j * L, L)]

            return jnp.sum(lax.fori_loop(0, GRP, rd, zero))

        def pass_a(i, acc):
            for u in range(UN):
                sl = pl.ds((i * UN + u) * L, L)
                el = (i * UN + u) * L + lane
                p = p_v[sl]
                t = t_v[sl]
                w = plsc.load_gather(w_v, [el // 3])
                sgn = jnp.where(p >= 0.0, jnp.float32(1.0), jnp.float32(-1.0))
                xa = jnp.abs(p)
                ya = t * sgn
                ratio = ya / jnp.maximum(xa, eps)
                bits = plsc.bitcast(ratio, jnp.int32)
                key = jnp.where(bits >= 0, bits, bits ^ jnp.int32(_MASK31))
                key_v[sl] = key
                wx_v[sl] = xa * w
                acc = acc + xa * w
            return acc

        tvec = lax.fori_loop(0, NV // UN, pass_a, zero)
        t_half = global_sum(tvec) * jnp.float32(0.5)

        def count_lt(q):
            qv = jnp.full((L,), q, jnp.int32)

            def body(i, acc):
                for u in range(UN):
                    sl = pl.ds((i * UN + u) * L, L)
                    kk = key_v[sl]
                    vv = wx_v[sl]
                    acc = acc + jnp.where(kk < qv, vv, jnp.float32(0.0))
                return acc

            return lax.fori_loop(0, NV // UN, body, zero)

        c0 = global_sum(count_lt(jnp.int32(0)))
        p_key = jnp.where(c0 < t_half, jnp.int32(0), jnp.int32(-2147483648))

        def round_body(r, p_key):
            q = p_key + (jnp.int32(1) << (30 - r))
            c = global_sum(count_lt(q))
            return jnp.where(c < t_half, q, p_key)

        p_key = lax.fori_loop(0, 31, round_body, p_key)

        pbits = jnp.where(p_key >= 0, p_key, p_key ^ jnp.int32(_MASK31))
        a_vec = plsc.bitcast(jnp.full((L,), pbits, jnp.int32), jnp.float32)

        def pass_c(i, acc):
            for u in range(UN):
                sl = pl.ds((i * UN + u) * L, L)
                el = (i * UN + u) * L + lane
                p = p_v[sl]
                t = t_v[sl]
                w = plsc.load_gather(w_v, [el // 3])
                acc = acc + w * jnp.abs(a_vec * p - t)
            return acc

        num_vec = lax.fori_loop(0, NV // UN, pass_c, zero)

        def pass_w(i, acc):
            return acc + w_v[pl.ds(i * L, L)]

        den_vec = lax.fori_loop(0, PCH // L, pass_w, zero)

        num = global_sum(num_vec)
        den = global_sum(den_vec)

        @pl.when(lid == 0)
        def _():
            out_v[...] = jnp.where(
                lane == 0, num, jnp.where(lane == 1, den, jnp.float32(0.0)))
            pltpu.sync_copy(out_v, out_hbm.at[pl.ds(b * L, L)])

    return k(pred_f, target_f, weight_f)


def kernel(pred, target, weight):
    pred_f = pred.reshape(B * M)
    target_f = target.reshape(B * M)
    weight_f = weight.reshape(B * N)
    out = _sc_point_loss(pred_f, target_f, weight_f).reshape(B, L)
    per_batch = out[:, 0]
    denom = 3.0 * jnp.maximum(out[:, 1], EPS)
    return jnp.mean(per_batch / denom)
